# baseline (speedup 1.0000x reference)
import jax
import jax.numpy as jnp
import numpy as np
from jax.experimental import pallas as pl

_B = 128
_N = 100000
_BC = 2048

_TINY = np.float32(np.finfo(np.float32).tiny)
_ONE = np.float32(1.0)
_KEY0 = np.uint32(0)
_KEY1 = np.uint32(42)


def _rotl(x, d):
    return (x << np.uint32(d)) | (x >> np.uint32(32 - d))


def _gumbel_argmax_kernel(x_ref, val_ref, idx_ref):
    k = pl.program_id(0)
    blk = x_ref[...]

    row = jax.lax.broadcasted_iota(jnp.uint32, blk.shape, 0)
    col = jax.lax.broadcasted_iota(jnp.int32, blk.shape, 1) + k * _BC
    cnt = row * np.uint32(_N) + col.astype(jnp.uint32)

    ks = (_KEY0, _KEY1, np.uint32(_KEY0 ^ _KEY1 ^ np.uint32(0x1BD11BDA)))
    rot = ((13, 15, 26, 6), (17, 29, 16, 24))
    x0 = jnp.full(blk.shape, ks[0], jnp.uint32)
    x1 = cnt + ks[1]
    for i in range(5):
        for r in rot[i % 2]:
            x0 = x0 + x1
            x1 = _rotl(x1, r)
            x1 = x0 ^ x1
        x0 = x0 + ks[(i + 1) % 3]
        x1 = x1 + ks[(i + 2) % 3] + np.uint32(i + 1)
    bits = x0 ^ x1

    fb = (bits >> np.uint32(9)) | np.uint32(0x3F800000)
    f = jax.lax.bitcast_convert_type(fb, jnp.float32) - _ONE
    u = jnp.maximum(_TINY, f * (_ONE - _TINY) + _TINY)
    g = -jnp.log(-jnp.log(u))

    m = jnp.where(col < _N, blk + g, -jnp.inf)
    rowmax = jnp.max(m, axis=1, keepdims=True)
    cand = jnp.where(m == rowmax, col, jnp.int32(np.iinfo(np.int32).max))
    rowarg = jnp.min(cand, axis=1, keepdims=True)

    @pl.when(k == 0)
    def _():
        val_ref[...] = rowmax
        idx_ref[...] = rowarg

    @pl.when(k != 0)
    def _():
        prev = val_ref[...]
        take = rowmax > prev
        val_ref[...] = jnp.where(take, rowmax, prev)
        idx_ref[...] = jnp.where(take, rowarg, idx_ref[...])


def kernel(logits):
    nb = pl.cdiv(_N, _BC)
    _, idx = pl.pallas_call(
        _gumbel_argmax_kernel,
        grid=(nb,),
        in_specs=[pl.BlockSpec((_B, _BC), lambda k: (0, k))],
        out_specs=[
            pl.BlockSpec((_B, 1), lambda k: (0, 0)),
            pl.BlockSpec((_B, 1), lambda k: (0, 0)),
        ],
        out_shape=[
            jax.ShapeDtypeStruct((_B, 1), jnp.float32),
            jax.ShapeDtypeStruct((_B, 1), jnp.int32),
        ],
    )(logits)
    return idx.astype(jnp.int64)
